# baseline (speedup 1.0000x reference)
import functools

import jax
import jax.numpy as jnp
from jax import lax
from jax.experimental import pallas as pl
from jax.experimental.pallas import tpu as pltpu
from jax.experimental.pallas import tpu_sc as plsc

B = 16384
D = 16
NC = 2
NS = 16
NW = NC * NS
BPW = B // NW
CB = 128
CHUNKS = BPW // CB


@functools.partial(
    pl.kernel,
    mesh=plsc.VectorSubcoreMesh(core_axis_name="c", subcore_axis_name="s"),
    out_type=jax.ShapeDtypeStruct((B,), jnp.float32),
    compiler_params=pltpu.CompilerParams(
        needs_layout_passes=False,
        use_tc_tiling_on_sc=False,
    ),
    scratch_types=[
        pltpu.VMEM((CHUNKS, CB), jnp.int32),
        pltpu.VMEM((CHUNKS, CB), jnp.int32),
        pltpu.VMEM((D, BPW), jnp.float32),
        pltpu.VMEM((D, BPW), jnp.float32),
        pltpu.VMEM((BPW,), jnp.float32),
        pltpu.SemaphoreType.DMA,
    ],
)
def _bpr_sc(lidx_hbm, iidx_hbm, lembT_hbm, iembT_hbm, out_hbm,
            lidx_v, iidx_v, lrows_v, irows_v, out_v, sem):
    wid = lax.axis_index("s") * NC + lax.axis_index("c")
    base = wid * BPW

    pltpu.sync_copy(lidx_hbm.at[wid], lidx_v)
    pltpu.sync_copy(iidx_hbm.at[wid], iidx_v)

    copies = []
    for f in range(D):
        for j in range(CHUNKS):
            copies.append(
                pltpu.async_copy(lembT_hbm.at[f].at[lidx_v.at[j]],
                                 lrows_v.at[f, pl.ds(j * CB, CB)], sem))
            copies.append(
                pltpu.async_copy(iembT_hbm.at[f].at[iidx_v.at[j]],
                                 irows_v.at[f, pl.ds(j * CB, CB)], sem))
    for c in copies:
        c.wait()

    def block(t, carry):
        r0 = t * 16
        acc = None
        for f in range(D):
            p = lrows_v[f, pl.ds(r0, 16)] * irows_v[f, pl.ds(r0, 16)]
            acc = p if acc is None else acc + p
        out_v[pl.ds(r0, 16)] = acc
        return carry

    lax.fori_loop(0, BPW // 16, block, 0)

    pltpu.sync_copy(out_v, out_hbm.at[pl.ds(base, BPW)])


def kernel(user_pos_indices, user_neg_indices, list_indices, item_indices,
           list_emb, item_emb):
    lidx = list_indices.astype(jnp.int32).reshape(NW, CHUNKS, CB)
    iidx = item_indices.astype(jnp.int32).reshape(NW, CHUNKS, CB)
    return _bpr_sc(lidx, iidx, list_emb.T, item_emb.T)


# speedup vs baseline: 18.4053x; 18.4053x over previous
import functools

import jax
import jax.numpy as jnp
from jax import lax
from jax.experimental import pallas as pl
from jax.experimental.pallas import tpu as pltpu
from jax.experimental.pallas import tpu_sc as plsc

B = 16384
D = 16
V = 1_000_000
NC = 2
NS = 16
NW = NC * NS
BPW = B // NW
CB = 128
CHUNKS = BPW // CB

SPAN = 244 * 128
MAIN = NW * SPAN
REM = 4 * 128
TAIL = V - MAIN - REM


@functools.partial(
    pl.kernel,
    mesh=plsc.VectorSubcoreMesh(core_axis_name="c", subcore_axis_name="s"),
    out_type=(jax.ShapeDtypeStruct((D * V,), jnp.float32),
              jax.ShapeDtypeStruct((D * V,), jnp.float32)),
    compiler_params=pltpu.CompilerParams(
        needs_layout_passes=False,
        use_tc_tiling_on_sc=True,
    ),
    scratch_types=[
        pltpu.VMEM((2 * SPAN,), jnp.float32),
        pltpu.VMEM((REM,), jnp.float32),
        pltpu.VMEM((TAIL,), jnp.float32),
        pltpu.SemaphoreType.DMA,
        pltpu.SemaphoreType.DMA,
    ],
)
def _detile_sc(lembT_hbm, iembT_hbm, ltail_hbm, itail_hbm,
               lout_hbm, iout_hbm, buf_v, rem_v, tail_v, rsem, wsem):
    wid = lax.axis_index("s") * NC + lax.axis_index("c")
    off = pl.multiple_of(wid * SPAN, 128)

    pending = [None, None]
    k = 0
    for src_hbm, dst_hbm in ((lembT_hbm, lout_hbm), (iembT_hbm, iout_hbm)):
        for f in range(D):
            slot = k % 2
            if pending[slot] is not None:
                pending[slot].wait()
            pltpu.async_copy(src_hbm.at[f].at[pl.ds(off, SPAN)],
                             buf_v.at[pl.ds(slot * SPAN, SPAN)], rsem).wait()
            pending[slot] = pltpu.async_copy(
                buf_v.at[pl.ds(slot * SPAN, SPAN)],
                dst_hbm.at[pl.ds(f * V + off, SPAN)], wsem)
            k += 1
    for p in pending:
        if p is not None:
            p.wait()

    for t, (src_hbm, tail_hbm, dst_hbm) in enumerate(
            ((lembT_hbm, ltail_hbm, lout_hbm),
             (iembT_hbm, itail_hbm, iout_hbm))):
        for f in range(D):
            @pl.when(wid == t * D + f)
            def _():
                pltpu.async_copy(src_hbm.at[f].at[pl.ds(MAIN, REM)],
                                 rem_v, rsem).wait()
                pltpu.sync_copy(tail_hbm.at[pl.ds(f * TAIL, TAIL)], tail_v)
                pltpu.sync_copy(rem_v, dst_hbm.at[pl.ds(f * V + MAIN, REM)])
                pltpu.sync_copy(tail_v,
                                dst_hbm.at[pl.ds(f * V + MAIN + REM, TAIL)])


@functools.partial(
    pl.kernel,
    mesh=plsc.VectorSubcoreMesh(core_axis_name="c", subcore_axis_name="s"),
    out_type=jax.ShapeDtypeStruct((B,), jnp.float32),
    compiler_params=pltpu.CompilerParams(
        needs_layout_passes=False,
        use_tc_tiling_on_sc=False,
    ),
    scratch_types=[
        pltpu.VMEM((CHUNKS, CB), jnp.int32),
        pltpu.VMEM((CHUNKS, CB), jnp.int32),
        pltpu.VMEM((D, BPW), jnp.float32),
        pltpu.VMEM((D, BPW), jnp.float32),
        pltpu.VMEM((BPW,), jnp.float32),
        pltpu.SemaphoreType.DMA,
    ],
)
def _bpr_sc(lidx_hbm, iidx_hbm, lembT_hbm, iembT_hbm, out_hbm,
            lidx_v, iidx_v, lrows_v, irows_v, out_v, sem):
    wid = lax.axis_index("s") * NC + lax.axis_index("c")
    base = wid * BPW

    pltpu.sync_copy(lidx_hbm.at[wid], lidx_v)
    pltpu.sync_copy(iidx_hbm.at[wid], iidx_v)

    copies = []
    for f in range(D):
        for j in range(CHUNKS):
            copies.append(
                pltpu.async_copy(lembT_hbm.at[f].at[lidx_v.at[j]],
                                 lrows_v.at[f, pl.ds(j * CB, CB)], sem))
            copies.append(
                pltpu.async_copy(iembT_hbm.at[f].at[iidx_v.at[j]],
                                 irows_v.at[f, pl.ds(j * CB, CB)], sem))
    for c in copies:
        c.wait()

    def block(t, carry):
        r0 = t * 16
        acc = None
        for f in range(D):
            p = lrows_v[f, pl.ds(r0, 16)] * irows_v[f, pl.ds(r0, 16)]
            acc = p if acc is None else acc + p
        out_v[pl.ds(r0, 16)] = acc
        return carry

    lax.fori_loop(0, BPW // 16, block, 0)

    pltpu.sync_copy(out_v, out_hbm.at[pl.ds(base, BPW)])


def kernel(user_pos_indices, user_neg_indices, list_indices, item_indices,
           list_emb, item_emb):
    lidx = list_indices.astype(jnp.int32).reshape(NW, CHUNKS, CB)
    iidx = item_indices.astype(jnp.int32).reshape(NW, CHUNKS, CB)
    ltail = list_emb[MAIN + REM:, :].T.reshape(D * TAIL)
    itail = item_emb[MAIN + REM:, :].T.reshape(D * TAIL)
    llin, ilin = _detile_sc(list_emb.T, item_emb.T, ltail, itail)
    return _bpr_sc(lidx, iidx, llin.reshape(D, V), ilin.reshape(D, V))


# speedup vs baseline: 19.0588x; 1.0355x over previous
import functools

import jax
import jax.numpy as jnp
from jax import lax
from jax.experimental import pallas as pl
from jax.experimental.pallas import tpu as pltpu
from jax.experimental.pallas import tpu_sc as plsc

B = 16384
D = 16
V = 1_000_000
NC = 2
NS = 16
NW = NC * NS
BPW = B // NW
CB = 128
CHUNKS = BPW // CB

SPAN = 244 * 128
MAIN = NW * SPAN
REM = 4 * 128
TAIL = V - MAIN - REM


@functools.partial(
    pl.kernel,
    mesh=plsc.VectorSubcoreMesh(core_axis_name="c", subcore_axis_name="s"),
    out_type=(jax.ShapeDtypeStruct((D * V,), jnp.float32),
              jax.ShapeDtypeStruct((D * V,), jnp.float32)),
    compiler_params=pltpu.CompilerParams(
        needs_layout_passes=False,
        use_tc_tiling_on_sc=True,
    ),
    scratch_types=[
        pltpu.VMEM((4 * SPAN,), jnp.float32),
        pltpu.VMEM((REM,), jnp.float32),
        pltpu.VMEM((TAIL,), jnp.float32),
        pltpu.SemaphoreType.DMA,
        pltpu.SemaphoreType.DMA,
    ],
)
def _detile_sc(lembT_hbm, iembT_hbm, ltail_hbm, itail_hbm,
               lout_hbm, iout_hbm, buf_v, rem_v, tail_v, rsem, wsem):
    wid = lax.axis_index("s") * NC + lax.axis_index("c")
    off = pl.multiple_of(wid * SPAN, 128)

    jobs = [(src, dst, f)
            for src, dst in ((lembT_hbm, lout_hbm), (iembT_hbm, iout_hbm))
            for f in range(D)]
    NSLOT = 4
    reads = [None] * NSLOT
    writes = [None] * NSLOT

    def slot_buf(slot):
        return buf_v.at[pl.ds(slot * SPAN, SPAN)]

    for k, (src_hbm, _, f) in enumerate(jobs[:NSLOT]):
        reads[k] = pltpu.async_copy(src_hbm.at[f].at[pl.ds(off, SPAN)],
                                    slot_buf(k), rsem)
    for k, (_, dst_hbm, f) in enumerate(jobs):
        slot = k % NSLOT
        reads[slot].wait()
        writes[slot] = pltpu.async_copy(
            slot_buf(slot), dst_hbm.at[pl.ds(f * V + off, SPAN)], wsem)
        kn = k + NSLOT
        if kn < len(jobs):
            writes[slot].wait()
            nsrc, _, nf = jobs[kn]
            reads[slot] = pltpu.async_copy(
                nsrc.at[nf].at[pl.ds(off, SPAN)], slot_buf(slot), rsem)
    for k in range(len(jobs) - NSLOT, len(jobs)):
        writes[k % NSLOT].wait()

    for t, (src_hbm, tail_hbm, dst_hbm) in enumerate(
            ((lembT_hbm, ltail_hbm, lout_hbm),
             (iembT_hbm, itail_hbm, iout_hbm))):
        for f in range(D):
            @pl.when(wid == t * D + f)
            def _():
                pltpu.async_copy(src_hbm.at[f].at[pl.ds(MAIN, REM)],
                                 rem_v, rsem).wait()
                pltpu.sync_copy(tail_hbm.at[pl.ds(f * TAIL, TAIL)], tail_v)
                pltpu.sync_copy(rem_v, dst_hbm.at[pl.ds(f * V + MAIN, REM)])
                pltpu.sync_copy(tail_v,
                                dst_hbm.at[pl.ds(f * V + MAIN + REM, TAIL)])


@functools.partial(
    pl.kernel,
    mesh=plsc.VectorSubcoreMesh(core_axis_name="c", subcore_axis_name="s"),
    out_type=jax.ShapeDtypeStruct((B,), jnp.float32),
    compiler_params=pltpu.CompilerParams(
        needs_layout_passes=False,
        use_tc_tiling_on_sc=False,
    ),
    scratch_types=[
        pltpu.VMEM((CHUNKS, CB), jnp.int32),
        pltpu.VMEM((CHUNKS, CB), jnp.int32),
        pltpu.VMEM((D, BPW), jnp.float32),
        pltpu.VMEM((D, BPW), jnp.float32),
        pltpu.VMEM((BPW,), jnp.float32),
        pltpu.SemaphoreType.DMA,
    ],
)
def _bpr_sc(lidx_hbm, iidx_hbm, lembT_hbm, iembT_hbm, out_hbm,
            lidx_v, iidx_v, lrows_v, irows_v, out_v, sem):
    wid = lax.axis_index("s") * NC + lax.axis_index("c")
    base = wid * BPW

    pltpu.sync_copy(lidx_hbm.at[wid], lidx_v)
    pltpu.sync_copy(iidx_hbm.at[wid], iidx_v)

    copies = []
    for f in range(D):
        for j in range(CHUNKS):
            copies.append(
                pltpu.async_copy(lembT_hbm.at[f].at[lidx_v.at[j]],
                                 lrows_v.at[f, pl.ds(j * CB, CB)], sem))
            copies.append(
                pltpu.async_copy(iembT_hbm.at[f].at[iidx_v.at[j]],
                                 irows_v.at[f, pl.ds(j * CB, CB)], sem))
    for c in copies:
        c.wait()

    def block(t, carry):
        r0 = t * 16
        acc = None
        for f in range(D):
            p = lrows_v[f, pl.ds(r0, 16)] * irows_v[f, pl.ds(r0, 16)]
            acc = p if acc is None else acc + p
        out_v[pl.ds(r0, 16)] = acc
        return carry

    lax.fori_loop(0, BPW // 16, block, 0)

    pltpu.sync_copy(out_v, out_hbm.at[pl.ds(base, BPW)])


def kernel(user_pos_indices, user_neg_indices, list_indices, item_indices,
           list_emb, item_emb):
    lidx = list_indices.astype(jnp.int32).reshape(NW, CHUNKS, CB)
    iidx = item_indices.astype(jnp.int32).reshape(NW, CHUNKS, CB)
    ltail = list_emb[MAIN + REM:, :].T.reshape(D * TAIL)
    itail = item_emb[MAIN + REM:, :].T.reshape(D * TAIL)
    llin, ilin = _detile_sc(list_emb.T, item_emb.T, ltail, itail)
    return _bpr_sc(lidx, iidx, llin.reshape(D, V), ilin.reshape(D, V))


# speedup vs baseline: 19.1787x; 1.0063x over previous
import functools

import jax
import jax.numpy as jnp
from jax import lax
from jax.experimental import pallas as pl
from jax.experimental.pallas import tpu as pltpu
from jax.experimental.pallas import tpu_sc as plsc

B = 16384
D = 16
V = 1_000_000
NC = 2
NS = 16
NW = NC * NS
BPW = B // NW
CB = 128
CHUNKS = BPW // CB

SPAN = 244 * 128
MAIN = NW * SPAN
REM = 4 * 128
TAIL = V - MAIN - REM


@functools.partial(
    pl.kernel,
    mesh=plsc.VectorSubcoreMesh(core_axis_name="c", subcore_axis_name="s"),
    out_type=(jax.ShapeDtypeStruct((D * V,), jnp.float32),
              jax.ShapeDtypeStruct((D * V,), jnp.float32)),
    compiler_params=pltpu.CompilerParams(
        needs_layout_passes=False,
        use_tc_tiling_on_sc=True,
    ),
    scratch_types=[
        pltpu.VMEM((4 * SPAN,), jnp.float32),
        pltpu.VMEM((REM,), jnp.float32),
        pltpu.VMEM((TAIL,), jnp.float32),
        pltpu.SemaphoreType.DMA,
        pltpu.SemaphoreType.DMA,
    ],
)
def _detile_sc(lembT_hbm, iembT_hbm, ltail_hbm, itail_hbm,
               lout_hbm, iout_hbm, buf_v, rem_v, tail_v, rsem, wsem):
    wid = lax.axis_index("s") * NC + lax.axis_index("c")
    off = pl.multiple_of(wid * SPAN, 128)

    HSPAN = SPAN // 2
    jobs = [(src, dst, f, sub)
            for src, dst in ((lembT_hbm, lout_hbm), (iembT_hbm, iout_hbm))
            for f in range(D)
            for sub in range(2)]
    NSLOT = 8
    reads = [None] * NSLOT
    writes = [None] * NSLOT

    def slot_buf(slot):
        return buf_v.at[pl.ds(slot * HSPAN, HSPAN)]

    def job_off(sub):
        return pl.multiple_of(off + sub * HSPAN, 128)

    for k, (src_hbm, _, f, sub) in enumerate(jobs[:NSLOT]):
        reads[k] = pltpu.async_copy(
            src_hbm.at[f].at[pl.ds(job_off(sub), HSPAN)], slot_buf(k), rsem)
    for k, (_, dst_hbm, f, sub) in enumerate(jobs):
        slot = k % NSLOT
        reads[slot].wait()
        writes[slot] = pltpu.async_copy(
            slot_buf(slot), dst_hbm.at[pl.ds(f * V + job_off(sub), HSPAN)],
            wsem)
        kn = k + NSLOT
        if kn < len(jobs):
            writes[slot].wait()
            nsrc, _, nf, nsub = jobs[kn]
            reads[slot] = pltpu.async_copy(
                nsrc.at[nf].at[pl.ds(job_off(nsub), HSPAN)],
                slot_buf(slot), rsem)
    for k in range(len(jobs) - NSLOT, len(jobs)):
        writes[k % NSLOT].wait()

    for t, (src_hbm, tail_hbm, dst_hbm) in enumerate(
            ((lembT_hbm, ltail_hbm, lout_hbm),
             (iembT_hbm, itail_hbm, iout_hbm))):
        for f in range(D):
            @pl.when(wid == t * D + f)
            def _():
                pltpu.async_copy(src_hbm.at[f].at[pl.ds(MAIN, REM)],
                                 rem_v, rsem).wait()
                pltpu.sync_copy(tail_hbm.at[pl.ds(f * TAIL, TAIL)], tail_v)
                pltpu.sync_copy(rem_v, dst_hbm.at[pl.ds(f * V + MAIN, REM)])
                pltpu.sync_copy(tail_v,
                                dst_hbm.at[pl.ds(f * V + MAIN + REM, TAIL)])


@functools.partial(
    pl.kernel,
    mesh=plsc.VectorSubcoreMesh(core_axis_name="c", subcore_axis_name="s"),
    out_type=jax.ShapeDtypeStruct((B,), jnp.float32),
    compiler_params=pltpu.CompilerParams(
        needs_layout_passes=False,
        use_tc_tiling_on_sc=False,
    ),
    scratch_types=[
        pltpu.VMEM((CHUNKS, CB), jnp.int32),
        pltpu.VMEM((CHUNKS, CB), jnp.int32),
        pltpu.VMEM((D, BPW), jnp.float32),
        pltpu.VMEM((D, BPW), jnp.float32),
        pltpu.VMEM((BPW,), jnp.float32),
        pltpu.SemaphoreType.DMA,
    ],
)
def _bpr_sc(lidx_hbm, iidx_hbm, lembT_hbm, iembT_hbm, out_hbm,
            lidx_v, iidx_v, lrows_v, irows_v, out_v, sem):
    wid = lax.axis_index("s") * NC + lax.axis_index("c")
    base = wid * BPW

    pltpu.sync_copy(lidx_hbm.at[wid], lidx_v)
    pltpu.sync_copy(iidx_hbm.at[wid], iidx_v)

    copies = []
    for f in range(D):
        for j in range(CHUNKS):
            copies.append(
                pltpu.async_copy(lembT_hbm.at[f].at[lidx_v.at[j]],
                                 lrows_v.at[f, pl.ds(j * CB, CB)], sem))
            copies.append(
                pltpu.async_copy(iembT_hbm.at[f].at[iidx_v.at[j]],
                                 irows_v.at[f, pl.ds(j * CB, CB)], sem))
    for c in copies:
        c.wait()

    def block(t, carry):
        r0 = t * 16
        acc = None
        for f in range(D):
            p = lrows_v[f, pl.ds(r0, 16)] * irows_v[f, pl.ds(r0, 16)]
            acc = p if acc is None else acc + p
        out_v[pl.ds(r0, 16)] = acc
        return carry

    lax.fori_loop(0, BPW // 16, block, 0)

    pltpu.sync_copy(out_v, out_hbm.at[pl.ds(base, BPW)])


def kernel(user_pos_indices, user_neg_indices, list_indices, item_indices,
           list_emb, item_emb):
    lidx = list_indices.astype(jnp.int32).reshape(NW, CHUNKS, CB)
    iidx = item_indices.astype(jnp.int32).reshape(NW, CHUNKS, CB)
    ltail = list_emb[MAIN + REM:, :].T.reshape(D * TAIL)
    itail = item_emb[MAIN + REM:, :].T.reshape(D * TAIL)
    llin, ilin = _detile_sc(list_emb.T, item_emb.T, ltail, itail)
    return _bpr_sc(lidx, iidx, llin.reshape(D, V), ilin.reshape(D, V))
